# baseline (speedup 1.0000x reference)
import functools

import jax
import jax.numpy as jnp
from jax import lax
from jax.experimental import pallas as pl
from jax.experimental.pallas import tpu as pltpu
from jax.experimental.pallas import tpu_sc as plsc

N_ROWS = 1000000
B_UPD = 16384
KK = 16
FF = 32
HH = 32
AA = 8

SC_NC = 2
SC_NS = 16
SC_NW = SC_NC * SC_NS

T_SWEEP = 8000
NT = N_ROWS // T_SWEEP
FL = T_SWEEP * AA // 128
T_UPD = 4096
CH = 128
B_PER_W = B_UPD // SC_NW

_NEG = -3.4028235e38
_IMAX = 2147483647


G = 16


def _sweep_body(v_ref, d_ref, s_ref, wv_ref, wd_ref, ws_ref, bh_ref,
                wo_ref, bo_ref, p_ref, logit_ref, mask_ref):
    v = v_ref[0]
    z = (jnp.dot(v, wv_ref[...], preferred_element_type=jnp.float32)
         + jnp.dot(d_ref[0], wd_ref[...], preferred_element_type=jnp.float32)
         + jnp.dot(s_ref[0], ws_ref[...], preferred_element_type=jnp.float32)
         + bh_ref[...])
    h = jnp.maximum(z, 0.0)
    logit_ref[0] = (
        jnp.dot(h, wo_ref[...], preferred_element_type=jnp.float32)
        + bo_ref[...])
    b = (v.astype(jnp.int32) == 2).astype(jnp.float32)
    mf = jnp.dot(b, p_ref[...], preferred_element_type=jnp.float32)
    mask_ref[0] = (mf > 0.5).astype(jnp.int8)


_sweep = pl.pallas_call(
    _sweep_body,
    grid=(NT,),
    in_specs=[
        pl.BlockSpec((1, FL, G * FF), lambda i: (i, 0, 0)),
        pl.BlockSpec((1, FL, G * KK * 3), lambda i: (i, 0, 0)),
        pl.BlockSpec((1, FL, G * KK), lambda i: (i, 0, 0)),
        pl.BlockSpec((G * FF, G * HH), lambda i: (0, 0)),
        pl.BlockSpec((G * KK * 3, G * HH), lambda i: (0, 0)),
        pl.BlockSpec((G * KK, G * HH), lambda i: (0, 0)),
        pl.BlockSpec((1, G * HH), lambda i: (0, 0)),
        pl.BlockSpec((G * HH, G * AA), lambda i: (0, 0)),
        pl.BlockSpec((1, G * AA), lambda i: (0, 0)),
        pl.BlockSpec((G * FF, G * AA), lambda i: (0, 0)),
    ],
    out_specs=[
        pl.BlockSpec((1, FL, 128), lambda i: (i, 0, 0)),
        pl.BlockSpec((1, FL, 128), lambda i: (i, 0, 0)),
    ],
    out_shape=[
        jax.ShapeDtypeStruct((NT, FL, 128), jnp.float32),
        jax.ShapeDtypeStruct((NT, FL, 128), jnp.int8),
    ],
)


def _upd_body(rows_ref, dk_ref, sk_ref, w_ref, bh_ref, wo_ref, bo_ref,
              out_ref):
    x = jnp.concatenate([rows_ref[...], dk_ref[...], sk_ref[...]], axis=1)
    z = jnp.dot(x, w_ref[...], preferred_element_type=jnp.float32)
    h = jnp.maximum(z + bh_ref[...], 0.0)
    out_ref[...] = (
        jnp.dot(h, wo_ref[...], preferred_element_type=jnp.float32)
        + bo_ref[...])


_upd = pl.pallas_call(
    _upd_body,
    grid=(B_UPD // T_UPD,),
    in_specs=[
        pl.BlockSpec((T_UPD, FF), lambda i: (i, 0)),
        pl.BlockSpec((T_UPD, KK * 3), lambda i: (i, 0)),
        pl.BlockSpec((T_UPD, KK), lambda i: (i, 0)),
        pl.BlockSpec((FF + KK * 3 + KK, HH), lambda i: (0, 0)),
        pl.BlockSpec((1, HH), lambda i: (0, 0)),
        pl.BlockSpec((HH, AA), lambda i: (0, 0)),
        pl.BlockSpec((1, AA), lambda i: (0, 0)),
    ],
    out_specs=[pl.BlockSpec((T_UPD, AA), lambda i: (i, 0))],
    out_shape=[jax.ShapeDtypeStruct((B_UPD, AA), jnp.float32)],
)


@functools.cache
def _sc_kernels():
    mesh = plsc.VectorSubcoreMesh(
        core_axis_name="c", subcore_axis_name="s",
        num_cores=SC_NC, num_subcores=SC_NS)

    @functools.partial(
        pl.kernel,
        out_type=jax.ShapeDtypeStruct((B_UPD, FF), jnp.float32),
        mesh=mesh,
        compiler_params=pltpu.CompilerParams(use_tc_tiling_on_sc=False),
        scratch_types=[
            pltpu.VMEM((CH,), jnp.int32),
            pltpu.VMEM((CH, FF), jnp.float32),
            pltpu.SemaphoreType.DMA,
        ],
    )
    def sc_gather(table_hbm, idx_hbm, out_hbm, idx_v, rows_v, sem):
        wid = lax.axis_index("s") * SC_NC + lax.axis_index("c")
        base = wid * B_PER_W
        for j in range(B_PER_W // CH):
            off = base + j * CH
            pltpu.sync_copy(idx_hbm.at[pl.ds(off, CH)], idx_v)
            pltpu.async_copy(table_hbm.at[idx_v], rows_v, sem).wait()
            pltpu.sync_copy(rows_v, out_hbm.at[pl.ds(off, CH)])

    @functools.partial(
        pl.kernel,
        out_type=(),
        mesh=mesh,
        compiler_params=pltpu.CompilerParams(use_tc_tiling_on_sc=False),
        scratch_types=[
            pltpu.VMEM((CH,), jnp.int32),
            pltpu.VMEM((CH, AA), jnp.float32),
            pltpu.SemaphoreType.DMA,
        ],
    )
    def sc_scatter(logits_hbm, idx_hbm, vals_hbm, idx_v, vals_v, sem):
        wid = lax.axis_index("s") * SC_NC + lax.axis_index("c")
        base = wid * B_PER_W
        for j in range(B_PER_W // CH):
            off = base + j * CH
            pltpu.sync_copy(idx_hbm.at[pl.ds(off, CH)], idx_v)
            pltpu.sync_copy(vals_hbm.at[pl.ds(off, CH)], vals_v)
            pltpu.async_copy(vals_v, logits_hbm.at[idx_v], sem).wait()

    return sc_gather, sc_scatter


def _fin_body(lg_ref, mk_ref, logp_ref, mp_ref, act_ref, fs, ii):
    p = pl.program_id(0)
    i = pl.program_id(1)

    @pl.when((p == 0) & (i == 0))
    def _():
        fs[0] = jnp.float32(_NEG)
        fs[1] = 0.0
        fs[2] = 0.0
        fs[3] = jnp.float32(_NEG)
        ii[0] = jnp.int32(_IMAX)

    l = lg_ref[...]
    ill = mk_ref[...] != 0

    @pl.when(p == 0)
    def _():
        m0 = fs[0]
        mn = jnp.maximum(m0, jnp.max(l))
        e = jnp.exp(l - mn)
        ts = jnp.sum(e)
        tsl = jnp.sum(jnp.where(ill, 0.0, e))
        scale = jnp.max(jnp.exp(jnp.full((8, 128), m0 - mn, jnp.float32)))
        fs[1] = fs[1] * scale + ts
        fs[2] = fs[2] * scale + tsl
        fs[0] = mn

        ml = jnp.where(ill, jnp.float32(_NEG), l)
        tb = jnp.max(ml)
        r = lax.broadcasted_iota(jnp.int32, (1, FL, 128), 1)
        c = lax.broadcasted_iota(jnp.int32, (1, FL, 128), 2)
        fi = (i * FL + r) * 128 + c
        tidx = jnp.min(jnp.where(ml == tb, fi, jnp.int32(_IMAX)))
        b0 = fs[3]
        i0 = ii[0]
        fs[3] = jnp.maximum(b0, tb)
        ii[0] = jnp.where(
            tb > b0, tidx,
            jnp.where(tb == b0, jnp.minimum(i0, tidx), i0))

    @pl.when(p == 1)
    def _():
        e = jnp.exp(l - fs[0])
        probs = e / fs[1]
        logp_ref[...] = jnp.where(ill, jnp.float32(-1e9),
                                  jnp.log(probs + 1e-8))
        mp_ref[...] = jnp.where(ill, 0.0, e / fs[2])

        @pl.when(i == 0)
        def _():
            act_ref[0, 0] = ii[0]


_fin = pl.pallas_call(
    _fin_body,
    grid=(2, NT),
    in_specs=[
        pl.BlockSpec((1, FL, 128), lambda p, i: (i, 0, 0)),
        pl.BlockSpec((1, FL, 128), lambda p, i: (i, 0, 0)),
    ],
    out_specs=[
        pl.BlockSpec((1, FL, 128), lambda p, i: (p * i, 0, 0)),
        pl.BlockSpec((1, FL, 128), lambda p, i: (p * i, 0, 0)),
        pl.BlockSpec(memory_space=pltpu.SMEM),
    ],
    out_shape=[
        jax.ShapeDtypeStruct((NT, FL, 128), jnp.float32),
        jax.ShapeDtypeStruct((NT, FL, 128), jnp.float32),
        jax.ShapeDtypeStruct((1, 1), jnp.int32),
    ],
    scratch_shapes=[
        pltpu.SMEM((4,), jnp.float32),
        pltpu.SMEM((1,), jnp.int32),
    ],
)


def kernel(V_features_local, diff_k_full, dist_k_full, vid_list, diff_k,
           dist_k, W_feat, W_diff, W_dist, b_hidden, W_out, b_out):
    dk_flat = diff_k.reshape(B_UPD, KK * 3)
    vid32 = vid_list.astype(jnp.int32)
    w_all = jnp.concatenate([W_feat, W_diff, W_dist], axis=0)
    bh2 = b_hidden.reshape(1, HH)
    bo2 = b_out.reshape(1, AA)

    vg = V_features_local.reshape(NT, FL, G * FF)
    dg = diff_k_full.reshape(NT, FL, G * KK * 3)
    sg = dist_k_full.reshape(NT, FL, G * KK)
    eye = jnp.eye(G, dtype=jnp.float32)
    wv = jnp.kron(eye, W_feat)
    wd = jnp.kron(eye, W_diff)
    ws = jnp.kron(eye, W_dist)
    wo = jnp.kron(eye, W_out)
    bh16 = jnp.tile(b_hidden, G).reshape(1, G * HH)
    bo16 = jnp.tile(b_out, G).reshape(1, G * AA)
    proj = jnp.kron(eye, jnp.eye(FF, AA, dtype=jnp.float32))

    sc_gather, sc_scatter = _sc_kernels()
    logits_f, mask_f = _sweep(vg, dg, sg, wv, wd, ws, bh16, wo, bo16, proj)
    rows = sc_gather(V_features_local, vid32)
    (new_logits,) = _upd(rows, dk_flat, dist_k, w_all, bh2, W_out, bo2)

    lref = jax.new_ref(logits_f.reshape(N_ROWS, AA))
    sc_scatter(lref, vid32, new_logits)
    logits1 = jax.freeze(lref)

    logp, mp, act = _fin(logits1.reshape(NT, FL, 128), mask_f)
    return (act.reshape(()), logp.reshape(-1), mp.reshape(-1))
